# baseline (speedup 1.0000x reference)
import functools

import jax
import jax.numpy as jnp
from jax import lax
from jax.experimental import pallas as pl
from jax.experimental.pallas import tpu as pltpu
from jax.experimental.pallas import tpu_sc as plsc

_NC = 2
_NS = 16
_NW = _NC * _NS

_D = 64
_SUB = 128
_CHUNK = 1024

_TB = 7936
_TB2 = _TB // 2
_NTB = 126
_TMAIN = _TB * _NTB
_TAIL = 64


def _in_copy(tab_hbm, scr, isem, j, s):
    return pltpu.make_async_copy(
        tab_hbm.at[:, pl.ds(j * _TB, _TB)], scr.at[s], isem.at[s])


def _out_copy(out_hbm, obuf, osem, j, s):
    return pltpu.make_async_copy(
        obuf.at[s], out_hbm.at[pl.ds(j * _TB2, _TB2)], osem.at[s])


def _pack_table_kernel(tab_hbm, tail_ref, out_hbm, scr, obuf, isem, osem):
    j = pl.program_id(0)
    s = j % 2

    @pl.when(j == 0)
    def _():
        _in_copy(tab_hbm, scr, isem, 0, 0).start()

    @pl.when(j < _NTB - 1)
    def _():
        _in_copy(tab_hbm, scr, isem, j + 1, (j + 1) % 2).start()

    _in_copy(tab_hbm, scr, isem, j, s).wait()

    @pl.when(j >= 2)
    def _():
        _out_copy(out_hbm, obuf, osem, j - 2, s).wait()

    eye = jnp.eye(_D, dtype=jnp.float32)
    t = lax.dot_general(scr[s], eye, (((0,), (0,)), ((), ())))
    obuf[s] = jnp.concatenate([t[:_TB2], t[_TB2:]], axis=1)
    _out_copy(out_hbm, obuf, osem, j, s).start()

    @pl.when(j == _NTB - 1)
    def _():
        _out_copy(out_hbm, obuf, osem, j - 1, (j - 1) % 2).wait()
        _out_copy(out_hbm, obuf, osem, j, s).wait()
        t2 = lax.dot_general(
            tail_ref[...], eye, (((0,), (0,)), ((), ())))
        obuf[0, 0:_TAIL // 2] = jnp.concatenate(
            [t2[:_TAIL // 2], t2[_TAIL // 2:]], axis=1)
        pltpu.make_async_copy(
            obuf.at[0].at[pl.ds(0, _TAIL // 2)],
            out_hbm.at[pl.ds(_TMAIN // 2, _TAIL // 2)],
            isem.at[0],
        ).start()
        pltpu.make_async_copy(
            obuf.at[0].at[pl.ds(0, _TAIL // 2)],
            out_hbm.at[pl.ds(_TMAIN // 2, _TAIL // 2)],
            isem.at[0],
        ).wait()


@functools.lru_cache(maxsize=None)
def _make_pack_table(v: int):
    assert v == _TMAIN + _TAIL
    return pl.pallas_call(
        _pack_table_kernel,
        grid=(_NTB,),
        in_specs=[
            pl.BlockSpec(memory_space=pltpu.MemorySpace.HBM),
            pl.BlockSpec(memory_space=pltpu.MemorySpace.VMEM),
        ],
        out_specs=pl.BlockSpec(memory_space=pltpu.MemorySpace.HBM),
        out_shape=jax.ShapeDtypeStruct((v // 2, 2 * _D), jnp.float32),
        scratch_shapes=[
            pltpu.VMEM((2, _D, _TB), jnp.float32),
            pltpu.VMEM((2, _TB2, 2 * _D), jnp.float32),
            pltpu.SemaphoreType.DMA((2,)),
            pltpu.SemaphoreType.DMA((2,)),
        ],
        compiler_params=pltpu.CompilerParams(
            dimension_semantics=("arbitrary",)),
    )


def _unpack_out_kernel(x_ref, o_ref):
    b2 = x_ref.shape[1]
    eye = jnp.eye(2 * _D, dtype=jnp.float32)
    t = lax.dot_general(eye, x_ref[0], (((1,), (1,)), ((), ())))
    o_ref[0, :, pl.ds(0, b2)] = t[:_D]
    o_ref[0, :, pl.ds(b2, b2)] = t[_D:]


@functools.lru_cache(maxsize=None)
def _make_unpack_out(b: int, h: int):
    return pl.pallas_call(
        _unpack_out_kernel,
        grid=(h,),
        in_specs=[pl.BlockSpec((1, b // 2, 2 * _D), lambda j: (j, 0, 0))],
        out_specs=pl.BlockSpec((1, _D, b), lambda j: (j, 0, 0)),
        out_shape=jax.ShapeDtypeStruct((h, _D, b), jnp.float32),
        compiler_params=pltpu.CompilerParams(
            dimension_semantics=("arbitrary",)),
    )


@functools.lru_cache(maxsize=None)
def _make_gather(n_total: int):
    assert n_total % _NW == 0
    per_w = n_total // _NW
    chunk = _CHUNK if per_w % _CHUNK == 0 else _CHUNK // 2
    assert per_w % chunk == 0
    n_chunks = per_w // chunk
    n_sub = chunk // _SUB

    mesh = plsc.VectorSubcoreMesh(core_axis_name="c", subcore_axis_name="s")

    @functools.partial(
        pl.kernel,
        out_type=jax.ShapeDtypeStruct((n_total, _D), jnp.float32),
        mesh=mesh,
        scratch_types=[
            pltpu.VMEM((per_w,), jnp.int32),
            pltpu.VMEM((chunk, _D), jnp.float32),
            pltpu.SemaphoreType.DMA,
        ],
        compiler_params=pltpu.CompilerParams(use_tc_tiling_on_sc=False),
    )
    def gather(idx_hbm, table_hbm, out_hbm, idx_v, rows_v, sem):
        wid = lax.axis_index("s") * _NC + lax.axis_index("c")
        base = wid * per_w
        pltpu.sync_copy(idx_hbm.at[pl.ds(base, per_w)], idx_v)

        @pl.loop(0, n_chunks)
        def _chunk(ci):
            off = pl.multiple_of(ci * chunk, chunk)
            cps = []
            for j in range(n_sub):
                cps.append(
                    pltpu.async_copy(
                        table_hbm.at[idx_v.at[pl.ds(off + j * _SUB, _SUB)]],
                        rows_v.at[pl.ds(j * _SUB, _SUB)],
                        sem,
                    )
                )
            for cp in cps:
                cp.wait()
            pltpu.sync_copy(rows_v, out_hbm.at[pl.ds(base + off, chunk)])

    return gather


def kernel(input_seq, embedding_weight):
    b, h = input_seq.shape
    n = b * h
    v = embedding_weight.shape[0]
    idx_hm = input_seq.T.astype(jnp.int32)
    r = jnp.arange(b, dtype=jnp.int32)
    bmap = (r % 2) * (b // 2) + r // 2
    idx = jnp.take(idx_hm, bmap, axis=1).reshape(n)
    j, rem = jnp.divmod(idx, _TB)
    e, p = jnp.divmod(rem, _TB2)
    c = idx - _TMAIN
    idx2 = jnp.where(
        idx < _TMAIN,
        j * _TB + 2 * p + e,
        _TMAIN + 2 * (c % (_TAIL // 2)) + c // (_TAIL // 2),
    )

    tab_t = embedding_weight.T
    tail = lax.slice(tab_t, (0, _TMAIN), (_D, v))
    packed = _make_pack_table(v)(tab_t, tail)
    tab = packed.reshape(v, _D)
    h2 = h // 2
    n2 = n // 2
    parts = []
    for i in range(2):
        idx_i = lax.slice(idx2, (i * n2,), ((i + 1) * n2,))
        out_i = _make_gather(n2)(idx_i, tab)
        o3_i = out_i.reshape(h2, b // 2, 2 * _D)
        o_phys_i = _make_unpack_out(b, h2)(o3_i)
        parts.append(o_phys_i)
    o_phys = jnp.concatenate(parts, axis=0)
    return o_phys.transpose((2, 0, 1))


# speedup vs baseline: 1.2020x; 1.2020x over previous
import functools

import jax
import jax.numpy as jnp
from jax import lax
from jax.experimental import pallas as pl
from jax.experimental.pallas import tpu as pltpu
from jax.experimental.pallas import tpu_sc as plsc

_NC = 2
_NS = 16
_NW = _NC * _NS

_D = 64
_SUB = 128
_CHUNK = 1024

_TB = 7936
_TB2 = _TB // 2
_NTB = 126
_TMAIN = _TB * _NTB
_TAIL = 64


def _in_copy(tab_hbm, scr, isem, j, s):
    return pltpu.make_async_copy(
        tab_hbm.at[:, pl.ds(j * _TB, _TB)], scr.at[s], isem.at[s])


def _out_copy(out_hbm, obuf, osem, j, s):
    return pltpu.make_async_copy(
        obuf.at[s], out_hbm.at[pl.ds(j * _TB2, _TB2)], osem.at[s])


def _pack_table_kernel(tab_hbm, tail_ref, out_hbm, scr, obuf, isem, osem):
    j = pl.program_id(0)
    s = j % 2

    @pl.when(j == 0)
    def _():
        _in_copy(tab_hbm, scr, isem, 0, 0).start()

    @pl.when(j < _NTB - 1)
    def _():
        _in_copy(tab_hbm, scr, isem, j + 1, (j + 1) % 2).start()

    _in_copy(tab_hbm, scr, isem, j, s).wait()

    @pl.when(j >= 2)
    def _():
        _out_copy(out_hbm, obuf, osem, j - 2, s).wait()

    eye = jnp.eye(_D, dtype=jnp.float32)
    t = lax.dot_general(scr[s], eye, (((0,), (0,)), ((), ())))
    obuf[s] = jnp.concatenate([t[:_TB2], t[_TB2:]], axis=1)
    _out_copy(out_hbm, obuf, osem, j, s).start()

    @pl.when(j == _NTB - 1)
    def _():
        _out_copy(out_hbm, obuf, osem, j - 1, (j - 1) % 2).wait()
        _out_copy(out_hbm, obuf, osem, j, s).wait()
        t2 = lax.dot_general(
            tail_ref[...], eye, (((0,), (0,)), ((), ())))
        obuf[0, 0:_TAIL // 2] = jnp.concatenate(
            [t2[:_TAIL // 2], t2[_TAIL // 2:]], axis=1)
        pltpu.make_async_copy(
            obuf.at[0].at[pl.ds(0, _TAIL // 2)],
            out_hbm.at[pl.ds(_TMAIN // 2, _TAIL // 2)],
            isem.at[0],
        ).start()
        pltpu.make_async_copy(
            obuf.at[0].at[pl.ds(0, _TAIL // 2)],
            out_hbm.at[pl.ds(_TMAIN // 2, _TAIL // 2)],
            isem.at[0],
        ).wait()


@functools.lru_cache(maxsize=None)
def _make_pack_table(v: int):
    assert v == _TMAIN + _TAIL
    return pl.pallas_call(
        _pack_table_kernel,
        grid=(_NTB,),
        in_specs=[
            pl.BlockSpec(memory_space=pltpu.MemorySpace.HBM),
            pl.BlockSpec(memory_space=pltpu.MemorySpace.VMEM),
        ],
        out_specs=pl.BlockSpec(memory_space=pltpu.MemorySpace.HBM),
        out_shape=jax.ShapeDtypeStruct((v // 2, 2 * _D), jnp.float32),
        scratch_shapes=[
            pltpu.VMEM((2, _D, _TB), jnp.float32),
            pltpu.VMEM((2, _TB2, 2 * _D), jnp.float32),
            pltpu.SemaphoreType.DMA((2,)),
            pltpu.SemaphoreType.DMA((2,)),
        ],
        compiler_params=pltpu.CompilerParams(
            dimension_semantics=("arbitrary",)),
    )


def _unpack_out_kernel(x_ref, *rest):
    o_ref = rest[-1]
    b2 = x_ref.shape[1]
    eye = jnp.eye(2 * _D, dtype=jnp.float32)
    t = lax.dot_general(eye, x_ref[0], (((1,), (1,)), ((), ())))
    o_ref[0, :, pl.ds(0, b2)] = t[:_D]
    o_ref[0, :, pl.ds(b2, b2)] = t[_D:]


@functools.lru_cache(maxsize=None)
def _make_unpack_out(b: int, h: int, h_total: int, h_off: int, aliased: bool):
    in_specs = [pl.BlockSpec((1, b // 2, 2 * _D), lambda j: (j, 0, 0))]
    if aliased:
        in_specs.append(pl.BlockSpec(memory_space=pltpu.MemorySpace.HBM))
    return pl.pallas_call(
        _unpack_out_kernel,
        grid=(h,),
        in_specs=in_specs,
        out_specs=pl.BlockSpec((1, _D, b), lambda j: (j + h_off, 0, 0)),
        out_shape=jax.ShapeDtypeStruct((h_total, _D, b), jnp.float32),
        input_output_aliases={1: 0} if aliased else {},
        compiler_params=pltpu.CompilerParams(
            dimension_semantics=("arbitrary",)),
    )


@functools.lru_cache(maxsize=None)
def _make_gather(n_total: int):
    assert n_total % _NW == 0
    per_w = n_total // _NW
    chunk = _CHUNK if per_w % _CHUNK == 0 else _CHUNK // 2
    assert per_w % chunk == 0
    n_chunks = per_w // chunk
    n_sub = chunk // _SUB

    mesh = plsc.VectorSubcoreMesh(core_axis_name="c", subcore_axis_name="s")

    @functools.partial(
        pl.kernel,
        out_type=jax.ShapeDtypeStruct((n_total, _D), jnp.float32),
        mesh=mesh,
        scratch_types=[
            pltpu.VMEM((per_w,), jnp.int32),
            pltpu.VMEM((chunk, _D), jnp.float32),
            pltpu.SemaphoreType.DMA,
        ],
        compiler_params=pltpu.CompilerParams(use_tc_tiling_on_sc=False),
    )
    def gather(idx_hbm, table_hbm, out_hbm, idx_v, rows_v, sem):
        wid = lax.axis_index("s") * _NC + lax.axis_index("c")
        base = wid * per_w
        pltpu.sync_copy(idx_hbm.at[pl.ds(base, per_w)], idx_v)

        @pl.loop(0, n_chunks)
        def _chunk(ci):
            off = pl.multiple_of(ci * chunk, chunk)
            cps = []
            for j in range(n_sub):
                cps.append(
                    pltpu.async_copy(
                        table_hbm.at[idx_v.at[pl.ds(off + j * _SUB, _SUB)]],
                        rows_v.at[pl.ds(j * _SUB, _SUB)],
                        sem,
                    )
                )
            for cp in cps:
                cp.wait()
            pltpu.sync_copy(rows_v, out_hbm.at[pl.ds(base + off, chunk)])

    return gather


def kernel(input_seq, embedding_weight):
    b, h = input_seq.shape
    n = b * h
    v = embedding_weight.shape[0]
    idx_hm = input_seq.T.astype(jnp.int32)
    r = jnp.arange(b, dtype=jnp.int32)
    bmap = (r % 2) * (b // 2) + r // 2
    idx = jnp.take(idx_hm, bmap, axis=1).reshape(n)
    j, rem = jnp.divmod(idx, _TB)
    e, p = jnp.divmod(rem, _TB2)
    c = idx - _TMAIN
    idx2 = jnp.where(
        idx < _TMAIN,
        j * _TB + 2 * p + e,
        _TMAIN + 2 * (c % (_TAIL // 2)) + c // (_TAIL // 2),
    )

    tab_t = embedding_weight.T
    tail = lax.slice(tab_t, (0, _TMAIN), (_D, v))
    packed = _make_pack_table(v)(tab_t, tail)
    tab = packed.reshape(v, _D)
    h2 = h // 2
    n2 = n // 2
    idx_a = lax.slice(idx2, (0,), (n2,))
    idx_b = lax.slice(idx2, (n2,), (n,))
    out_a = _make_gather(n2)(idx_a, tab)
    out_b = _make_gather(n2)(idx_b, tab)
    o3_a = out_a.reshape(h2, b // 2, 2 * _D)
    o3_b = out_b.reshape(h2, b // 2, 2 * _D)
    part = _make_unpack_out(b, h2, h, 0, False)(o3_a)
    o_phys = _make_unpack_out(b, h2, h, h2, True)(o3_b, part)
    return o_phys.transpose((2, 0, 1))
